# baseline (speedup 1.0000x reference)
import functools

import jax
import jax.numpy as jnp
from jax import lax
from jax.experimental import pallas as pl
from jax.experimental.pallas import tpu as pltpu
from jax.experimental.pallas import tpu_sc as plsc

CH = 128
KB = 2
BT = 1024
DP = 128


def _sc_gather_body(NC, N_CH, D,
                    idx_hbm, tab, out, idx_v, buf, tbuf, sem0, sem1):
    cid = lax.axis_index("c")
    sid = lax.axis_index("s")
    wid = sid * NC + cid
    rows_pw = N_CH * CH
    gch = KB * CH
    n_half = N_CH // KB // 2

    pltpu.sync_copy(idx_hbm.at[pl.ds(wid * N_CH, N_CH)], idx_v)

    iota = lax.iota(jnp.int32, 16)
    sems = (sem0, sem1)

    def copies(j, half):
        sem = sems[half]
        return [
            pltpu.make_async_copy(tab.at[idx_v.at[j * KB + b]],
                                  buf.at[pl.ds(half * gch + b * CH, CH)],
                                  sem)
            for b in range(KB)
        ]

    def fire(j, half):
        for c in copies(j, half):
            c.start()

    def drain_transpose_store(j, half):
        for c in copies(j, half):
            c.wait()
        base = half * gch

        def tr(g, carry2):
            rows = g * 16 + iota
            for c0 in range(D):
                cols = jnp.bitwise_and(c0 + iota, D - 1)
                vals = plsc.load_gather(buf, [base + rows, cols])
                plsc.store_scatter(tbuf, [cols, rows], vals)
            return carry2
        lax.fori_loop(0, gch // 16, tr, 0)

        pltpu.sync_copy(
            tbuf, out.at[:, pl.ds(wid * rows_pw + j * gch, gch)])

    fire(0, 0)

    def body(j2, carry):
        j0 = j2 * 2
        fire(j0 + 1, 1)
        drain_transpose_store(j0, 0)

        @pl.when(j2 + 1 < n_half)
        def _():
            fire(j0 + 2, 0)

        drain_transpose_store(j0 + 1, 1)
        return carry
    lax.fori_loop(0, n_half, body, 0)


PW = 256


def _sc_pack_body(NC, NW, V, D, tab_t, tail_rows, out, tin, tout, sem0, sem1):
    cid = lax.axis_index("c")
    sid = lax.axis_index("s")
    wid = sid * NC + cid

    nfull = (V // PW) // NW
    n_blk = jnp.where(wid == NW - 1, (V // PW) - nfull * (NW - 1), nfull)
    blk0 = wid * nfull
    n_half = n_blk // 2

    iota = lax.iota(jnp.int32, 16)
    sems = (sem0, sem1)

    def cp(i, half):
        return pltpu.make_async_copy(
            tab_t.at[:, pl.ds((blk0 + i) * PW, PW)],
            tin.at[:, pl.ds(half * PW, PW)], sems[half])

    def work(i, half):
        cp(i, half).wait()
        base = half * PW

        def tr(g, carry2):
            rows = g * 16 + iota
            for c0 in range(D):
                cols = jnp.bitwise_and(c0 + iota, D - 1)
                vals = plsc.load_gather(tin, [cols, base + rows])
                plsc.store_scatter(tout, [rows, cols], vals)
            return carry2
        lax.fori_loop(0, PW // 16, tr, 0)
        pltpu.sync_copy(tout, out.at[pl.ds((blk0 + i) * PW, PW)])

    cp(0, 0).start()

    def body(j2, carry):
        i0 = j2 * 2
        cp(i0 + 1, 1).start()
        work(i0, 0)

        @pl.when(j2 + 1 < n_half)
        def _():
            cp(i0 + 2, 0).start()

        work(i0 + 1, 1)
        return carry
    lax.fori_loop(0, n_half, body, 0)

    tail = V - (V // PW) * PW

    @pl.when(wid == NW - 1)
    def _():
        pltpu.sync_copy(tail_rows, tout.at[pl.ds(0, tail)])
        pltpu.sync_copy(tout.at[pl.ds(0, tail)],
                        out.at[pl.ds(V - tail, tail)])


def _tc_loss_body(B, *refs):
    wv_refs = refs[0:6]
    cv_refs = refs[6:16]
    out_ref = refs[16]

    def renorm(x):
        sos = jnp.sum(x * x, axis=0, keepdims=True)
        n = jnp.sqrt(sos)
        return x * jnp.minimum(1.0, 1.0 / jnp.maximum(n, 1e-7))

    cont = renorm(cv_refs[0][:])
    for r in cv_refs[1:]:
        cont = cont + renorm(r[:])
    cont = cont * 0.1

    cent = renorm(wv_refs[0][:])
    pos = jnp.sum(cont * cent, axis=0)
    acc = jnp.sum(jnp.log1p(jnp.exp(-pos))) * (1.0 / B)
    for k in range(1, 6):
        dk = jnp.sum(cont * renorm(wv_refs[k][:]), axis=0)
        acc = acc + jnp.sum(jnp.log1p(jnp.exp(dk))) * (1.0 / (5.0 * B))

    prev = jnp.where(pl.program_id(0) == 0, 0.0, out_ref[0, 0])
    out_ref[0, 0] = prev + acc


def kernel(context, center, negcase, wordvec, contextvec):
    B = center.shape[0]
    D = wordvec.shape[1]

    eye_p = jnp.eye(D, DP, dtype=jnp.float32)
    cv_p = contextvec @ eye_p

    info = plsc.get_sparse_core_info()
    NC, NS = info.num_cores, info.num_subcores
    NW = NC * NS

    cen = center.astype(jnp.int32).reshape(-1)
    neg = negcase.astype(jnp.int32).T.reshape(-1)
    ctx = context.astype(jnp.int32).T.reshape(-1)
    wv_idx = jnp.concatenate([cen, neg]).reshape(-1, CH)
    cv_idx = ctx.reshape(-1, CH)

    WV_CH = (6 * B) // (NW * CH)
    CV_CH = (10 * B) // (NW * CH)

    mesh = plsc.VectorSubcoreMesh(core_axis_name="c", subcore_axis_name="s")
    def make_gather(n_rows, n_ch):
        return functools.partial(
            pl.kernel,
            mesh=mesh,
            out_type=jax.ShapeDtypeStruct((D, n_rows), jnp.float32),
            scratch_types=[
                pltpu.VMEM((n_ch, CH), jnp.int32),
                pltpu.VMEM((2 * KB * CH, DP), jnp.float32),
                pltpu.VMEM((D, KB * CH), jnp.float32),
                pltpu.SemaphoreType.DMA,
                pltpu.SemaphoreType.DMA,
            ],
            compiler_params=pltpu.CompilerParams(
                use_tc_tiling_on_sc=True, needs_layout_passes=False),
        )(functools.partial(_sc_gather_body, NC, n_ch, D))

    V = wordvec.shape[0]
    sc_pack = functools.partial(
        pl.kernel,
        mesh=mesh,
        out_type=jax.ShapeDtypeStruct((V, DP), jnp.float32),
        scratch_types=[
            pltpu.VMEM((D, 2 * PW), jnp.float32),
            pltpu.VMEM((PW, DP), jnp.float32),
            pltpu.SemaphoreType.DMA,
            pltpu.SemaphoreType.DMA,
        ],
        compiler_params=pltpu.CompilerParams(
            use_tc_tiling_on_sc=True, needs_layout_passes=False),
    )(functools.partial(_sc_pack_body, NC, NW, V, D))

    n_tail = V - (V // PW) * PW
    wv_tail = wordvec[V - n_tail:] @ eye_p
    wv_p = sc_pack(wordvec.T, wv_tail)
    wv_rows = make_gather(6 * B, WV_CH)(wv_idx, wv_p)
    cv_rows = make_gather(10 * B, CV_CH)(cv_idx, cv_p)

    grid = B // BT
    in_specs = (
        [pl.BlockSpec((D, BT), lambda i, r=r: (0, r * grid + i))
         for r in range(6)]
        + [pl.BlockSpec((D, BT), lambda i, r=r: (0, r * grid + i))
           for r in range(10)]
    )
    out = pl.pallas_call(
        functools.partial(_tc_loss_body, B),
        grid=(grid,),
        in_specs=in_specs,
        out_specs=pl.BlockSpec(memory_space=pltpu.SMEM),
        out_shape=jax.ShapeDtypeStruct((1, 1), jnp.float32),
    )(*([wv_rows] * 6 + [cv_rows] * 10))
    return out[0, 0]


# speedup vs baseline: 1.2632x; 1.2632x over previous
import functools

import jax
import jax.numpy as jnp
from jax import lax
from jax.experimental import pallas as pl
from jax.experimental.pallas import tpu as pltpu
from jax.experimental.pallas import tpu_sc as plsc

CH = 128
KB = 2
BT = 1024
DP = 128


def _sc_gather_body(NC, N_CH, D,
                    idx_hbm, rm_hbm, tab, out, idx_v, rm_v, buf, tbuf,
                    sem0, sem1):
    cid = lax.axis_index("c")
    sid = lax.axis_index("s")
    wid = sid * NC + cid
    rows_pw = N_CH * CH
    gch = KB * CH
    n_half = N_CH // KB // 2

    pltpu.sync_copy(idx_hbm.at[pl.ds(wid * N_CH, N_CH)], idx_v)
    pltpu.sync_copy(rm_hbm.at[pl.ds(wid * N_CH, N_CH)], rm_v)

    iota = lax.iota(jnp.int32, 16)
    sems = (sem0, sem1)

    def copies(j, half):
        sem = sems[half]
        return [
            pltpu.make_async_copy(tab.at[idx_v.at[j * KB + b]],
                                  buf.at[pl.ds(half * gch + b * CH, CH)],
                                  sem)
            for b in range(KB)
        ]

    def fire(j, half):
        for c in copies(j, half):
            c.start()

    def drain_transpose_store(j, half):
        for c in copies(j, half):
            c.wait()
        base = half * gch

        def tr(g, carry2):
            rows = g * 16 + iota
            rmrow = jnp.full((16,), j * KB + g // 8, jnp.int32)
            rmcol = (g % 8) * 16 + iota
            offs = plsc.load_gather(rm_v, [rmrow, rmcol]) * 32
            for c0 in range(D):
                cols = jnp.bitwise_and(c0 + iota, D - 1)
                vals = plsc.load_gather(buf, [base + rows, offs + cols])
                plsc.store_scatter(tbuf, [cols, rows], vals)
            return carry2
        lax.fori_loop(0, gch // 16, tr, 0)

        pltpu.sync_copy(
            tbuf, out.at[:, pl.ds(wid * rows_pw + j * gch, gch)])

    fire(0, 0)

    def body(j2, carry):
        j0 = j2 * 2
        fire(j0 + 1, 1)
        drain_transpose_store(j0, 0)

        @pl.when(j2 + 1 < n_half)
        def _():
            fire(j0 + 2, 0)

        drain_transpose_store(j0 + 1, 1)
        return carry
    lax.fori_loop(0, n_half, body, 0)


PW = 256


def _sc_pack_body(NC, NW, V, D, tab_t, tail_rows, out, tin, tout, sem0, sem1):
    cid = lax.axis_index("c")
    sid = lax.axis_index("s")
    wid = sid * NC + cid

    nfull = (V // PW) // NW
    n_blk = jnp.where(wid == NW - 1, (V // PW) - nfull * (NW - 1), nfull)
    blk0 = wid * nfull
    n_half = n_blk // 2

    iota = lax.iota(jnp.int32, 16)
    sems = (sem0, sem1)

    def cp(i, half):
        return pltpu.make_async_copy(
            tab_t.at[:, pl.ds((blk0 + i) * PW, PW)],
            tin.at[:, pl.ds(half * PW, PW)], sems[half])

    def work(i, half):
        cp(i, half).wait()
        base = half * PW

        def tr(g, carry2):
            rows = g * 16 + iota
            orow = rows // 4
            ocol0 = (rows % 4) * 32
            for c0 in range(D):
                cols = jnp.bitwise_and(c0 + iota, D - 1)
                vals = plsc.load_gather(tin, [cols, base + rows])
                plsc.store_scatter(tout, [orow, ocol0 + cols], vals)
            return carry2
        lax.fori_loop(0, PW // 16, tr, 0)
        pltpu.sync_copy(tout, out.at[pl.ds((blk0 + i) * (PW // 4), PW // 4)])

    cp(0, 0).start()

    def body(j2, carry):
        i0 = j2 * 2
        cp(i0 + 1, 1).start()
        work(i0, 0)

        @pl.when(j2 + 1 < n_half)
        def _():
            cp(i0 + 2, 0).start()

        work(i0 + 1, 1)
        return carry
    lax.fori_loop(0, n_half, body, 0)

    tail = (V - (V // PW) * PW) // 4

    @pl.when(wid == NW - 1)
    def _():
        pltpu.sync_copy(tail_rows, tout.at[pl.ds(0, tail)])
        pltpu.sync_copy(tout.at[pl.ds(0, tail)],
                        out.at[pl.ds(V // 4 - tail, tail)])


def _tc_loss_body(B, *refs):
    wv_refs = refs[0:6]
    cv_refs = refs[6:16]
    out_ref = refs[16]

    def renorm(x):
        sos = jnp.sum(x * x, axis=0, keepdims=True)
        n = jnp.sqrt(sos)
        return x * jnp.minimum(1.0, 1.0 / jnp.maximum(n, 1e-7))

    cont = renorm(cv_refs[0][:])
    for r in cv_refs[1:]:
        cont = cont + renorm(r[:])
    cont = cont * 0.1

    cent = renorm(wv_refs[0][:])
    pos = jnp.sum(cont * cent, axis=0)
    acc = jnp.sum(jnp.log1p(jnp.exp(-pos))) * (1.0 / B)
    for k in range(1, 6):
        dk = jnp.sum(cont * renorm(wv_refs[k][:]), axis=0)
        acc = acc + jnp.sum(jnp.log1p(jnp.exp(dk))) * (1.0 / (5.0 * B))

    prev = jnp.where(pl.program_id(0) == 0, 0.0, out_ref[0, 0])
    out_ref[0, 0] = prev + acc


def kernel(context, center, negcase, wordvec, contextvec):
    B = center.shape[0]
    D = wordvec.shape[1]

    eye_p = jnp.eye(D, DP, dtype=jnp.float32)
    cv_p = contextvec @ eye_p

    info = plsc.get_sparse_core_info()
    NC, NS = info.num_cores, info.num_subcores
    NW = NC * NS

    cen = center.astype(jnp.int32).reshape(-1)
    neg = negcase.astype(jnp.int32).T.reshape(-1)
    ctx = context.astype(jnp.int32).T.reshape(-1)
    wv_flat = jnp.concatenate([cen, neg])
    wv_idx = (wv_flat // 4).reshape(-1, CH)
    wv_rm = (wv_flat % 4).reshape(-1, CH)
    cv_idx = ctx.reshape(-1, CH)
    cv_rm = jnp.zeros_like(cv_idx)

    WV_CH = (6 * B) // (NW * CH)
    CV_CH = (10 * B) // (NW * CH)

    mesh = plsc.VectorSubcoreMesh(core_axis_name="c", subcore_axis_name="s")
    def make_gather(n_rows, n_ch):
        return functools.partial(
            pl.kernel,
            mesh=mesh,
            out_type=jax.ShapeDtypeStruct((D, n_rows), jnp.float32),
            scratch_types=[
                pltpu.VMEM((n_ch, CH), jnp.int32),
                pltpu.VMEM((n_ch, CH), jnp.int32),
                pltpu.VMEM((2 * KB * CH, DP), jnp.float32),
                pltpu.VMEM((D, KB * CH), jnp.float32),
                pltpu.SemaphoreType.DMA,
                pltpu.SemaphoreType.DMA,
            ],
            compiler_params=pltpu.CompilerParams(
                use_tc_tiling_on_sc=True, needs_layout_passes=False),
        )(functools.partial(_sc_gather_body, NC, n_ch, D))

    V = wordvec.shape[0]
    sc_pack = functools.partial(
        pl.kernel,
        mesh=mesh,
        out_type=jax.ShapeDtypeStruct((V // 4, DP), jnp.float32),
        scratch_types=[
            pltpu.VMEM((D, 2 * PW), jnp.float32),
            pltpu.VMEM((PW // 4, DP), jnp.float32),
            pltpu.SemaphoreType.DMA,
            pltpu.SemaphoreType.DMA,
        ],
        compiler_params=pltpu.CompilerParams(
            use_tc_tiling_on_sc=True, needs_layout_passes=False),
    )(functools.partial(_sc_pack_body, NC, NW, V, D))

    n_tail = V - (V // PW) * PW
    wv_tail = jnp.reshape(wordvec[V - n_tail:], (n_tail // 4, DP))
    wv_p = sc_pack(wordvec.T, wv_tail)
    wv_rows = make_gather(6 * B, WV_CH)(wv_idx, wv_rm, wv_p)
    cv_rows = make_gather(10 * B, CV_CH)(cv_idx, cv_rm, cv_p)

    grid = B // BT
    in_specs = (
        [pl.BlockSpec((D, BT), lambda i, r=r: (0, r * grid + i))
         for r in range(6)]
        + [pl.BlockSpec((D, BT), lambda i, r=r: (0, r * grid + i))
           for r in range(10)]
    )
    out = pl.pallas_call(
        functools.partial(_tc_loss_body, B),
        grid=(grid,),
        in_specs=in_specs,
        out_specs=pl.BlockSpec(memory_space=pltpu.SMEM),
        out_shape=jax.ShapeDtypeStruct((1, 1), jnp.float32),
    )(*([wv_rows] * 6 + [cv_rows] * 10))
    return out[0, 0]
